# baseline (speedup 1.0000x reference)
import functools

import jax
import jax.numpy as jnp
from jax.experimental import pallas as pl
from jax.experimental.pallas import tpu as pltpu


def _lrelu(v):
    return jnp.where(v > 0, v, 0.01 * v)


def _lin0_body(x_ref, w_ref, b_ref, o_ref):
    o_ref[...] = _lrelu(
        jnp.dot(x_ref[...], w_ref[...], preferred_element_type=jnp.float32)
        + b_ref[...])


def _encode_nodes(x, w, b):
    n, _ = x.shape
    dim = w.shape[1]
    return pl.pallas_call(
        _lin0_body,
        out_shape=jax.ShapeDtypeStruct((n, dim), jnp.float32),
    )(x, w, b.reshape(1, dim))


def _eh_body(ea_ref, w_ref, b_ref, o_ref):
    o_ref[...] = _lrelu(
        jnp.dot(ea_ref[...], w_ref[...], preferred_element_type=jnp.float32)
        + b_ref[...])


def _encode_edges(edge_attr, w, b, te):
    e, four = edge_attr.shape
    dim = w.shape[1]
    grid = e // te
    return pl.pallas_call(
        _eh_body,
        grid=(grid,),
        in_specs=[
            pl.BlockSpec((te, four), lambda i: (i, 0)),
            pl.BlockSpec((four, dim), lambda i: (0, 0)),
            pl.BlockSpec((1, dim), lambda i: (0, 0)),
        ],
        out_specs=pl.BlockSpec((te, dim), lambda i: (i, 0)),
        out_shape=jax.ShapeDtypeStruct((e, dim), jnp.float32),
    )(edge_attr, w, b.reshape(1, dim))


def _msg_body(u_ref, eh_ref, w2d_ref, b2_ref, o_ref, *, dim):
    u = u_ref[...]
    g = jnp.dot(u, w2d_ref[...], preferred_element_type=jnp.float32)
    acc = jnp.dot(u, b2_ref[...], preferred_element_type=jnp.float32)
    eh = eh_ref[...]
    for k in range(dim):
        acc = acc + eh[:, k:k + 1] * g[:, k * dim:(k + 1) * dim]
    o_ref[...] = acc


def _messages(u, eh, w2d, b2, te):
    e, dim = u.shape
    grid = e // te
    return pl.pallas_call(
        functools.partial(_msg_body, dim=dim),
        grid=(grid,),
        in_specs=[
            pl.BlockSpec((te, dim), lambda i: (i, 0)),
            pl.BlockSpec((te, dim), lambda i: (i, 0)),
            pl.BlockSpec((dim, dim * dim), lambda i: (0, 0)),
            pl.BlockSpec((dim, dim), lambda i: (0, 0)),
        ],
        out_specs=pl.BlockSpec((te, dim), lambda i: (i, 0)),
        out_shape=jax.ShapeDtypeStruct((e, dim), jnp.float32),
    )(u, eh, w2d, b2)


def _gru_body(out_ref, h_ref, aggr_ref, dinv_ref, convw_ref, convb_ref,
              wih_t_ref, whh_t_ref, bih_ref, bhh_ref, o_ref, h2_ref, *, dim):
    out = out_ref[...]
    h = h_ref[...]
    aggr = aggr_ref[...] * dinv_ref[...]
    m = _lrelu(
        jnp.dot(out, convw_ref[...], preferred_element_type=jnp.float32)
        + aggr + convb_ref[...])
    gi = jnp.dot(m, wih_t_ref[...], preferred_element_type=jnp.float32) + bih_ref[...]
    gh = jnp.dot(h, whh_t_ref[...], preferred_element_type=jnp.float32) + bhh_ref[...]
    ir, iz, inn = gi[:, :dim], gi[:, dim:2 * dim], gi[:, 2 * dim:]
    hr, hz, hn = gh[:, :dim], gh[:, dim:2 * dim], gh[:, 2 * dim:]
    r = jax.nn.sigmoid(ir + hr)
    z = jax.nn.sigmoid(iz + hz)
    nn = jnp.tanh(inn + r * hn)
    hnew = (1.0 - z) * nn + z * h
    o_ref[...] = hnew
    h2_ref[...] = hnew


def _gru(out, h, aggr, dinv, convw, convb, wih_t, whh_t, bih, bhh):
    n, dim = out.shape
    return pl.pallas_call(
        functools.partial(_gru_body, dim=dim),
        out_shape=(jax.ShapeDtypeStruct((n, dim), jnp.float32),
                   jax.ShapeDtypeStruct((n, dim), jnp.float32)),
    )(out, h, aggr, dinv, convw, convb, wih_t, whh_t,
      bih.reshape(1, -1), bhh.reshape(1, -1))


def _readout_body(gs_ref, gj_ref, s1_ref, s1b_ref, s2_ref, s2b_ref,
                  j1blk_ref, j1b_ref, j2half_ref, j2b_ref,
                  stem_ref, jb_ref):
    hs = _lrelu(
        jnp.dot(gs_ref[...], s1_ref[...], preferred_element_type=jnp.float32)
        + s1b_ref[...])
    stem_ref[...] = (
        jnp.dot(hs, s2_ref[...], preferred_element_type=jnp.float32)
        + s2b_ref[...])
    hj = _lrelu(
        jnp.dot(gj_ref[...], j1blk_ref[...], preferred_element_type=jnp.float32)
        + j1b_ref[...])
    jb_ref[...] = (
        jnp.dot(hj, j2half_ref[...], preferred_element_type=jnp.float32)
        + j2b_ref[...])


def _readouts(gs, gj, s1, s1b, s2, s2b, j1blk, j1b2, j2half, j2b):
    nstem = gs.shape[0]
    njb = gj.shape[0]
    nout = s2.shape[1]
    return pl.pallas_call(
        _readout_body,
        out_shape=(jax.ShapeDtypeStruct((nstem, nout), jnp.float32),
                   jax.ShapeDtypeStruct((njb, 1), jnp.float32)),
    )(gs, gj, s1, s1b.reshape(1, -1), s2, s2b.reshape(1, -1),
      j1blk, j1b2.reshape(1, -1), j2half, j2b.reshape(1, 1))


def _set2set_body(out_ref, batch_ref, gsum_ref, lwq_ref, lwr_ref, lb_ref,
                  so_ref, *, nb, dim):
    out = out_ref[...]
    gates = gsum_ref[...]
    i_ = jax.nn.sigmoid(gates[:, :dim])
    g_ = jnp.tanh(gates[:, 2 * dim:3 * dim])
    o_ = jax.nn.sigmoid(gates[:, 3 * dim:])
    cl = i_ * g_
    qvec = o_ * jnp.tanh(cl)
    e = jnp.dot(out, qvec.T, preferred_element_type=jnp.float32)
    bvec = batch_ref[...]
    iota = jax.lax.broadcasted_iota(jnp.int32, (1, nb), 1)
    oh_bool = bvec == iota
    oh = oh_bool.astype(jnp.float32)
    neg = jnp.float32(-1e30)
    masked = jnp.where(oh_bool, e, neg)
    emax = jnp.max(masked, axis=0, keepdims=True)
    emax = jnp.where(emax < -1e29, 0.0, emax)
    e_shift = e - jnp.dot(oh, emax.T, preferred_element_type=jnp.float32)
    a = jnp.exp(e_shift)
    asum = jnp.sum(oh * a, axis=0, keepdims=True)
    asum = jnp.clip(asum, 1e-12, None)
    a = a / jnp.dot(oh, asum.T, preferred_element_type=jnp.float32)
    ao = a * out
    rvec = jax.lax.dot_general(oh, ao, (((0,), (0,)), ((), ())),
                               preferred_element_type=jnp.float32)
    so_ref[...] = (
        jnp.dot(rvec, lwr_ref[...], preferred_element_type=jnp.float32)
        + jnp.dot(qvec, lwq_ref[...], preferred_element_type=jnp.float32)
        + lb_ref[...])


def _set2set(out, batch, gsum, lwq, lwr, lb, nb):
    n, dim = out.shape
    return pl.pallas_call(
        functools.partial(_set2set_body, nb=nb, dim=dim),
        out_shape=jax.ShapeDtypeStruct((nb, 2), jnp.float32),
    )(out, batch.reshape(n, 1), gsum.reshape(1, -1), lwq, lwr,
      lb.reshape(1, 2))


def kernel(x, edge_attr, params, edge_index, stem_atmidx, jbond_atmidx, batch):
    p = params
    n, _ = x.shape
    e = edge_attr.shape[0]
    dim = p['lin0_w'].shape[1]
    nb = 256

    src = edge_index[0]
    dst = edge_index[1]

    w2d = (p['en2_w'].reshape(dim, dim, dim).transpose(1, 0, 2)
           .reshape(dim, dim * dim))
    b2 = p['en2_b'].reshape(dim, dim)
    wih_t = p['gru_wih'].T
    whh_t = p['gru_whh'].T
    j1 = p['j1_w']
    j1blk = jnp.block([[j1, jnp.zeros_like(j1)], [jnp.zeros_like(j1), j1]])
    j1b2 = jnp.concatenate([p['j1_b'], p['j1_b']])
    j2half = 0.5 * jnp.concatenate([p['j2_w'], p['j2_w']], axis=0)
    lwq = p['lout_w'][:dim]
    lwr = p['lout_w'][dim:]
    gsum = p['lstm_bih'] + p['lstm_bhh']

    te = 2000 if e % 2000 == 0 else e

    out = _encode_nodes(x, p['lin0_w'], p['lin0_b'])
    h = out
    eh = _encode_edges(edge_attr, p['en1_w'], p['en1_b'], min(te * 10, e))

    cnt = jax.ops.segment_sum(jnp.ones((e,), jnp.float32), dst, num_segments=n)
    dinv = (1.0 / jnp.clip(cnt, 1.0, None)).reshape(n, 1)

    for _ in range(6):
        u = out[src]
        msg = _messages(u, eh, w2d, b2, te)
        aggr = jax.ops.segment_sum(msg, dst, num_segments=n)
        out, h = _gru(out, h, aggr, dinv, p['conv_root'],
                      p['conv_b'].reshape(1, dim), wih_t, whh_t,
                      p['gru_bih'], p['gru_bhh'])

    gth = out[jnp.concatenate([stem_atmidx, jbond_atmidx.reshape(-1)])]
    gs = gth[:stem_atmidx.shape[0]]
    gj = gth[stem_atmidx.shape[0]:].reshape(-1, 2 * dim)

    stem_preds, jb = _readouts(gs, gj, p['s1_w'], p['s1_b'], p['s2_w'],
                               p['s2_b'], j1blk, j1b2, j2half, p['j2_b'])
    jbond_preds = jb.reshape(-1)

    scalar_outs = _set2set(out, batch, gsum, lwq, lwr, p['lout_b'], nb)
    return scalar_outs, stem_preds, jbond_preds


# speedup vs baseline: 1.3624x; 1.3624x over previous
import functools

import jax
import jax.numpy as jnp
from jax import lax
from jax.experimental import pallas as pl
from jax.experimental.pallas import tpu as pltpu
from jax.experimental.pallas import tpu_sc as plsc

_NC, _NS = 2, 16
_NW = _NC * _NS


def _lrelu(v):
    return jnp.where(v > 0, v, 0.01 * v)


def _sc_mesh():
    return plsc.VectorSubcoreMesh(core_axis_name="c", subcore_axis_name="s",
                                  num_cores=_NC, num_subcores=_NS)


def _sc_gather(table, idx, chunk):
    g = idx.shape[0]
    dim = table.shape[1]
    per_w = g // _NW
    nchunks = per_w // chunk
    assert per_w % chunk == 0 and g % _NW == 0

    @functools.partial(
        pl.kernel, mesh=_sc_mesh(),
        out_type=jax.ShapeDtypeStruct((g, dim), jnp.float32),
        compiler_params=pltpu.CompilerParams(use_tc_tiling_on_sc=False),
        scratch_types=[pltpu.VMEM((chunk,), jnp.int32),
                       pltpu.VMEM((chunk, dim), jnp.float32),
                       pltpu.SemaphoreType.DMA],
    )
    def k(table_hbm, idx_hbm, out_hbm, idx_v, rows_v, sem):
        wid = lax.axis_index("s") * _NC + lax.axis_index("c")
        for c in range(nchunks):
            base = wid * per_w + c * chunk
            pltpu.sync_copy(idx_hbm.at[pl.ds(base, chunk)], idx_v)
            pltpu.async_copy(table_hbm.at[idx_v], rows_v, sem).wait()
            pltpu.sync_copy(rows_v, out_hbm.at[pl.ds(base, chunk)])

    return k(table, idx)


def _sc_scatter_add(vals, idx, n, chunk):
    e, dim = vals.shape
    per_w = e // _NW
    nchunks = per_w // chunk
    rows_per_tile = n // _NS
    assert per_w % chunk == 0 and e % _NW == 0 and n % _NS == 0

    @functools.partial(
        pl.kernel, mesh=_sc_mesh(),
        out_type=jax.ShapeDtypeStruct((_NC, n, dim), jnp.float32),
        compiler_params=pltpu.CompilerParams(use_tc_tiling_on_sc=False),
        scratch_types=[pltpu.VMEM((chunk,), jnp.int32),
                       pltpu.VMEM((chunk, dim), jnp.float32),
                       pltpu.VMEM_SHARED((n, dim), jnp.float32),
                       pltpu.SemaphoreType.DMA],
    )
    def k(vals_hbm, idx_hbm, zeros_hbm, out_hbm, idx_v, rows_v, acc_sh, sem):
        cid = lax.axis_index("c")
        sid = lax.axis_index("s")
        wid = sid * _NC + cid
        r0 = sid * rows_per_tile
        pltpu.sync_copy(zeros_hbm.at[pl.ds(r0, rows_per_tile)],
                        acc_sh.at[pl.ds(r0, rows_per_tile)])
        plsc.subcore_barrier()
        for c in range(nchunks):
            base = wid * per_w + c * chunk
            pltpu.sync_copy(idx_hbm.at[pl.ds(base, chunk)], idx_v)
            pltpu.sync_copy(vals_hbm.at[pl.ds(base, chunk)], rows_v)
            pltpu.sync_copy(rows_v, acc_sh.at[idx_v], add=True)
        plsc.subcore_barrier()
        pltpu.sync_copy(acc_sh.at[pl.ds(r0, rows_per_tile)],
                        out_hbm.at[cid].at[pl.ds(r0, rows_per_tile)])

    zeros = jnp.zeros((n, dim), jnp.float32)
    return k(vals, idx, zeros)


def _lin0_body(x_ref, w_ref, b_ref, o_ref):
    o_ref[...] = _lrelu(
        jnp.dot(x_ref[...], w_ref[...], preferred_element_type=jnp.float32)
        + b_ref[...])


def _encode_nodes(x, w, b):
    n, _ = x.shape
    dim = w.shape[1]
    return pl.pallas_call(
        _lin0_body,
        out_shape=jax.ShapeDtypeStruct((n, dim), jnp.float32),
    )(x, w, b.reshape(1, dim))


def _eh_body(ea_ref, w_ref, b_ref, o_ref):
    o_ref[...] = _lrelu(
        jnp.dot(ea_ref[...], w_ref[...], preferred_element_type=jnp.float32)
        + b_ref[...])


def _encode_edges(edge_attr, w, b, te):
    e, four = edge_attr.shape
    dim = w.shape[1]
    grid = e // te
    return pl.pallas_call(
        _eh_body,
        grid=(grid,),
        in_specs=[
            pl.BlockSpec((te, four), lambda i: (i, 0)),
            pl.BlockSpec((four, dim), lambda i: (0, 0)),
            pl.BlockSpec((1, dim), lambda i: (0, 0)),
        ],
        out_specs=pl.BlockSpec((te, dim), lambda i: (i, 0)),
        out_shape=jax.ShapeDtypeStruct((e, dim), jnp.float32),
    )(edge_attr, w, b.reshape(1, dim))


def _msg_body(u_ref, eh_ref, w2d_ref, b2_ref, o_ref, *, dim):
    u = u_ref[...]
    g = jnp.dot(u, w2d_ref[...], preferred_element_type=jnp.float32)
    acc = jnp.dot(u, b2_ref[...], preferred_element_type=jnp.float32)
    eh = eh_ref[...]
    for k in range(dim):
        acc = acc + eh[:, k:k + 1] * g[:, k * dim:(k + 1) * dim]
    o_ref[...] = acc


def _messages(u, eh, w2d, b2, te):
    e, dim = u.shape
    grid = e // te
    return pl.pallas_call(
        functools.partial(_msg_body, dim=dim),
        grid=(grid,),
        in_specs=[
            pl.BlockSpec((te, dim), lambda i: (i, 0)),
            pl.BlockSpec((te, dim), lambda i: (i, 0)),
            pl.BlockSpec((dim, dim * dim), lambda i: (0, 0)),
            pl.BlockSpec((dim, dim), lambda i: (0, 0)),
        ],
        out_specs=pl.BlockSpec((te, dim), lambda i: (i, 0)),
        out_shape=jax.ShapeDtypeStruct((e, dim), jnp.float32),
    )(u, eh, w2d, b2)


def _gru_body(out_ref, h_ref, a0_ref, a1_ref, c0_ref, c1_ref,
              convw_ref, convb_ref,
              wih_t_ref, whh_t_ref, bih_ref, bhh_ref, o_ref, h2_ref, *, dim):
    out = out_ref[...]
    h = h_ref[...]
    dinv = 1.0 / jnp.clip(c0_ref[...] + c1_ref[...], 1.0, None)
    aggr = (a0_ref[...] + a1_ref[...]) * dinv
    m = _lrelu(
        jnp.dot(out, convw_ref[...], preferred_element_type=jnp.float32)
        + aggr + convb_ref[...])
    gi = jnp.dot(m, wih_t_ref[...], preferred_element_type=jnp.float32) + bih_ref[...]
    gh = jnp.dot(h, whh_t_ref[...], preferred_element_type=jnp.float32) + bhh_ref[...]
    ir, iz, inn = gi[:, :dim], gi[:, dim:2 * dim], gi[:, 2 * dim:]
    hr, hz, hn = gh[:, :dim], gh[:, dim:2 * dim], gh[:, 2 * dim:]
    r = jax.nn.sigmoid(ir + hr)
    z = jax.nn.sigmoid(iz + hz)
    nn = jnp.tanh(inn + r * hn)
    hnew = (1.0 - z) * nn + z * h
    o_ref[...] = hnew
    h2_ref[...] = hnew


def _gru(out, h, ap, cp, convw, convb, wih_t, whh_t, bih, bhh):
    n, dim = out.shape
    return pl.pallas_call(
        functools.partial(_gru_body, dim=dim),
        out_shape=(jax.ShapeDtypeStruct((n, dim), jnp.float32),
                   jax.ShapeDtypeStruct((n, dim), jnp.float32)),
    )(out, h, ap[0], ap[1], cp[0], cp[1], convw, convb, wih_t, whh_t,
      bih.reshape(1, -1), bhh.reshape(1, -1))


def _readout_body(gs_ref, gj_ref, s1_ref, s1b_ref, s2_ref, s2b_ref,
                  j1blk_ref, j1b_ref, j2half_ref, j2b_ref,
                  stem_ref, jb_ref):
    hs = _lrelu(
        jnp.dot(gs_ref[...], s1_ref[...], preferred_element_type=jnp.float32)
        + s1b_ref[...])
    stem_ref[...] = (
        jnp.dot(hs, s2_ref[...], preferred_element_type=jnp.float32)
        + s2b_ref[...])
    hj = _lrelu(
        jnp.dot(gj_ref[...], j1blk_ref[...], preferred_element_type=jnp.float32)
        + j1b_ref[...])
    jb_ref[...] = (
        jnp.dot(hj, j2half_ref[...], preferred_element_type=jnp.float32)
        + j2b_ref[...])


def _readouts(gs, gj, s1, s1b, s2, s2b, j1blk, j1b2, j2half, j2b):
    nstem = gs.shape[0]
    njb = gj.shape[0]
    nout = s2.shape[1]
    return pl.pallas_call(
        _readout_body,
        out_shape=(jax.ShapeDtypeStruct((nstem, nout), jnp.float32),
                   jax.ShapeDtypeStruct((njb, 1), jnp.float32)),
    )(gs, gj, s1, s1b.reshape(1, -1), s2, s2b.reshape(1, -1),
      j1blk, j1b2.reshape(1, -1), j2half, j2b.reshape(1, 1))


def _set2set_body(out_ref, batch_ref, gsum_ref, lwq_ref, lwr_ref, lb_ref,
                  so_ref, *, nb, dim):
    out = out_ref[...]
    gates = gsum_ref[...]
    i_ = jax.nn.sigmoid(gates[:, :dim])
    g_ = jnp.tanh(gates[:, 2 * dim:3 * dim])
    o_ = jax.nn.sigmoid(gates[:, 3 * dim:])
    cl = i_ * g_
    qvec = o_ * jnp.tanh(cl)
    e = jnp.dot(out, qvec.T, preferred_element_type=jnp.float32)
    bvec = batch_ref[...]
    iota = jax.lax.broadcasted_iota(jnp.int32, (1, nb), 1)
    oh_bool = bvec == iota
    oh = oh_bool.astype(jnp.float32)
    neg = jnp.float32(-1e30)
    masked = jnp.where(oh_bool, e, neg)
    emax = jnp.max(masked, axis=0, keepdims=True)
    emax = jnp.where(emax < -1e29, 0.0, emax)
    e_shift = e - jnp.dot(oh, emax.T, preferred_element_type=jnp.float32)
    a = jnp.exp(e_shift)
    asum = jnp.sum(oh * a, axis=0, keepdims=True)
    asum = jnp.clip(asum, 1e-12, None)
    a = a / jnp.dot(oh, asum.T, preferred_element_type=jnp.float32)
    ao = a * out
    rvec = jax.lax.dot_general(oh, ao, (((0,), (0,)), ((), ())),
                               preferred_element_type=jnp.float32)
    so_ref[...] = (
        jnp.dot(rvec, lwr_ref[...], preferred_element_type=jnp.float32)
        + jnp.dot(qvec, lwq_ref[...], preferred_element_type=jnp.float32)
        + lb_ref[...])


def _set2set(out, batch, gsum, lwq, lwr, lb, nb):
    n, dim = out.shape
    return pl.pallas_call(
        functools.partial(_set2set_body, nb=nb, dim=dim),
        out_shape=jax.ShapeDtypeStruct((nb, 2), jnp.float32),
    )(out, batch.reshape(n, 1), gsum.reshape(1, -1), lwq, lwr,
      lb.reshape(1, 2))


def kernel(x, edge_attr, params, edge_index, stem_atmidx, jbond_atmidx, batch):
    p = params
    n, _ = x.shape
    e = edge_attr.shape[0]
    dim = p['lin0_w'].shape[1]
    nb = 256

    src = edge_index[0]
    dst = edge_index[1]

    w2d = (p['en2_w'].reshape(dim, dim, dim).transpose(1, 0, 2)
           .reshape(dim, dim * dim))
    b2 = p['en2_b'].reshape(dim, dim)
    wih_t = p['gru_wih'].T
    whh_t = p['gru_whh'].T
    j1 = p['j1_w']
    j1blk = jnp.block([[j1, jnp.zeros_like(j1)], [jnp.zeros_like(j1), j1]])
    j1b2 = jnp.concatenate([p['j1_b'], p['j1_b']])
    j2half = 0.5 * jnp.concatenate([p['j2_w'], p['j2_w']], axis=0)
    lwq = p['lout_w'][:dim]
    lwr = p['lout_w'][dim:]
    gsum = p['lstm_bih'] + p['lstm_bhh']

    te = 2000 if e % 2000 == 0 else e

    out = _encode_nodes(x, p['lin0_w'], p['lin0_b'])
    h = out
    eh = _encode_edges(edge_attr, p['en1_w'], p['en1_b'], min(te * 10, e))

    cp = _sc_scatter_add(jnp.ones((e, dim), jnp.float32), dst, n, 1000)

    for _ in range(6):
        u = _sc_gather(out, src, 1000)
        msg = _messages(u, eh, w2d, b2, te)
        ap = _sc_scatter_add(msg, dst, n, 1000)
        out, h = _gru(out, h, ap, cp, p['conv_root'],
                      p['conv_b'].reshape(1, dim), wih_t, whh_t,
                      p['gru_bih'], p['gru_bhh'])

    gth = _sc_gather(
        out, jnp.concatenate([stem_atmidx, jbond_atmidx.reshape(-1)]), 128)
    gs = gth[:stem_atmidx.shape[0]]
    gj = gth[stem_atmidx.shape[0]:].reshape(-1, 2 * dim)

    stem_preds, jb = _readouts(gs, gj, p['s1_w'], p['s1_b'], p['s2_w'],
                               p['s2_b'], j1blk, j1b2, j2half, p['j2_b'])
    jbond_preds = jb.reshape(-1)

    scalar_outs = _set2set(out, batch, gsum, lwq, lwr, p['lout_b'], nb)
    return scalar_outs, stem_preds, jbond_preds


# speedup vs baseline: 3.7590x; 2.7592x over previous
import functools

import jax
import jax.numpy as jnp
from jax import lax
from jax.experimental import pallas as pl
from jax.experimental.pallas import tpu as pltpu
from jax.experimental.pallas import tpu_sc as plsc

_NC, _NS = 2, 16
_NW = _NC * _NS


def _lrelu(v):
    return jnp.where(v > 0, v, 0.01 * v)


def _sc_mesh():
    return plsc.VectorSubcoreMesh(core_axis_name="c", subcore_axis_name="s",
                                  num_cores=_NC, num_subcores=_NS)


def _sc_gather(table, idx, chunk):
    g = idx.shape[0]
    dim = table.shape[1]
    per_w = g // _NW
    nchunks = per_w // chunk
    assert per_w % chunk == 0 and g % _NW == 0

    @functools.partial(
        pl.kernel, mesh=_sc_mesh(),
        out_type=jax.ShapeDtypeStruct((g, dim), jnp.float32),
        compiler_params=pltpu.CompilerParams(use_tc_tiling_on_sc=False),
        scratch_types=[pltpu.VMEM((chunk,), jnp.int32),
                       pltpu.VMEM((chunk, dim), jnp.float32),
                       pltpu.SemaphoreType.DMA],
    )
    def k(table_hbm, idx_hbm, out_hbm, idx_v, rows_v, sem):
        wid = lax.axis_index("s") * _NC + lax.axis_index("c")
        for c in range(nchunks):
            base = wid * per_w + c * chunk
            pltpu.sync_copy(idx_hbm.at[pl.ds(base, chunk)], idx_v)
            pltpu.async_copy(table_hbm.at[idx_v], rows_v, sem).wait()
            pltpu.sync_copy(rows_v, out_hbm.at[pl.ds(base, chunk)])

    return k(table, idx)


def _sc_scatter_add(vals, idx, n, chunk):
    e, dim = vals.shape
    per_w = e // _NW
    nchunks = per_w // chunk
    rows_per_tile = n // _NS
    assert per_w % chunk == 0 and e % _NW == 0 and n % _NS == 0

    @functools.partial(
        pl.kernel, mesh=_sc_mesh(),
        out_type=jax.ShapeDtypeStruct((_NC, n, dim), jnp.float32),
        compiler_params=pltpu.CompilerParams(use_tc_tiling_on_sc=False),
        scratch_types=[pltpu.VMEM((chunk,), jnp.int32),
                       pltpu.VMEM((chunk, dim), jnp.float32),
                       pltpu.VMEM_SHARED((n, dim), jnp.float32),
                       pltpu.SemaphoreType.DMA],
    )
    def k(vals_hbm, idx_hbm, zeros_hbm, out_hbm, idx_v, rows_v, acc_sh, sem):
        cid = lax.axis_index("c")
        sid = lax.axis_index("s")
        wid = sid * _NC + cid
        r0 = sid * rows_per_tile
        pltpu.sync_copy(zeros_hbm.at[pl.ds(r0, rows_per_tile)],
                        acc_sh.at[pl.ds(r0, rows_per_tile)])
        plsc.subcore_barrier()
        for c in range(nchunks):
            base = wid * per_w + c * chunk
            pltpu.sync_copy(idx_hbm.at[pl.ds(base, chunk)], idx_v)
            pltpu.sync_copy(vals_hbm.at[pl.ds(base, chunk)], rows_v)
            pltpu.sync_copy(rows_v, acc_sh.at[idx_v], add=True)
        plsc.subcore_barrier()
        pltpu.sync_copy(acc_sh.at[pl.ds(r0, rows_per_tile)],
                        out_hbm.at[cid].at[pl.ds(r0, rows_per_tile)])

    zeros = jnp.zeros((n, dim), jnp.float32)
    return k(vals, idx, zeros)


def _lin0_body(x_ref, w_ref, b_ref, o_ref):
    o_ref[...] = _lrelu(
        jnp.dot(x_ref[...], w_ref[...], preferred_element_type=jnp.float32)
        + b_ref[...])


def _encode_nodes(x, w, b):
    n, _ = x.shape
    dim = w.shape[1]
    return pl.pallas_call(
        _lin0_body,
        out_shape=jax.ShapeDtypeStruct((n, dim), jnp.float32),
    )(x, w, b.reshape(1, dim))


def _eh_body(ea_ref, w_ref, b_ref, o_ref):
    o_ref[...] = _lrelu(
        jnp.dot(ea_ref[...], w_ref[...], preferred_element_type=jnp.float32)
        + b_ref[...])


def _encode_edges(edge_attr, w, b, te):
    e, four = edge_attr.shape
    dim = w.shape[1]
    grid = e // te
    return pl.pallas_call(
        _eh_body,
        grid=(grid,),
        in_specs=[
            pl.BlockSpec((te, four), lambda i: (i, 0)),
            pl.BlockSpec((four, dim), lambda i: (0, 0)),
            pl.BlockSpec((1, dim), lambda i: (0, 0)),
        ],
        out_specs=pl.BlockSpec((te, dim), lambda i: (i, 0)),
        out_shape=jax.ShapeDtypeStruct((e, dim), jnp.float32),
    )(edge_attr, w, b.reshape(1, dim))


def _msg_body(u_ref, eh_ref, rexp_ref, tile_ref, w2r_ref, b2_ref, o_ref, *,
              dim):
    u = u_ref[...]
    ehx = jnp.dot(eh_ref[...], rexp_ref[...],
                  preferred_element_type=jnp.float32)
    utile = jnp.dot(u, tile_ref[...], preferred_element_type=jnp.float32)
    z = ehx * utile
    o_ref[...] = (
        jnp.dot(z, w2r_ref[...], preferred_element_type=jnp.float32)
        + jnp.dot(u, b2_ref[...], preferred_element_type=jnp.float32))


def _messages(u, eh, rexp, tmat, w2r, b2, te):
    e, dim = u.shape
    grid = e // te
    return pl.pallas_call(
        functools.partial(_msg_body, dim=dim),
        grid=(grid,),
        in_specs=[
            pl.BlockSpec((te, dim), lambda i: (i, 0)),
            pl.BlockSpec((te, dim), lambda i: (i, 0)),
            pl.BlockSpec((dim, dim * dim), lambda i: (0, 0)),
            pl.BlockSpec((dim, dim * dim), lambda i: (0, 0)),
            pl.BlockSpec((dim * dim, dim), lambda i: (0, 0)),
            pl.BlockSpec((dim, dim), lambda i: (0, 0)),
        ],
        out_specs=pl.BlockSpec((te, dim), lambda i: (i, 0)),
        out_shape=jax.ShapeDtypeStruct((e, dim), jnp.float32),
    )(u, eh, rexp, tmat, w2r, b2)


def _gru_body(out_ref, h_ref, a0_ref, a1_ref, c0_ref, c1_ref,
              convw_ref, convb_ref,
              wih_t_ref, whh_t_ref, bih_ref, bhh_ref, o_ref, h2_ref, *, dim):
    out = out_ref[...]
    h = h_ref[...]
    dinv = 1.0 / jnp.clip(c0_ref[...] + c1_ref[...], 1.0, None)
    aggr = (a0_ref[...] + a1_ref[...]) * dinv
    m = _lrelu(
        jnp.dot(out, convw_ref[...], preferred_element_type=jnp.float32)
        + aggr + convb_ref[...])
    gi = jnp.dot(m, wih_t_ref[...], preferred_element_type=jnp.float32) + bih_ref[...]
    gh = jnp.dot(h, whh_t_ref[...], preferred_element_type=jnp.float32) + bhh_ref[...]
    ir, iz, inn = gi[:, :dim], gi[:, dim:2 * dim], gi[:, 2 * dim:]
    hr, hz, hn = gh[:, :dim], gh[:, dim:2 * dim], gh[:, 2 * dim:]
    r = jax.nn.sigmoid(ir + hr)
    z = jax.nn.sigmoid(iz + hz)
    nn = jnp.tanh(inn + r * hn)
    hnew = (1.0 - z) * nn + z * h
    o_ref[...] = hnew
    h2_ref[...] = hnew


def _gru(out, h, ap, cp, convw, convb, wih_t, whh_t, bih, bhh):
    n, dim = out.shape
    return pl.pallas_call(
        functools.partial(_gru_body, dim=dim),
        out_shape=(jax.ShapeDtypeStruct((n, dim), jnp.float32),
                   jax.ShapeDtypeStruct((n, dim), jnp.float32)),
    )(out, h, ap[0], ap[1], cp[0], cp[1], convw, convb, wih_t, whh_t,
      bih.reshape(1, -1), bhh.reshape(1, -1))


def _readout_body(gs_ref, gj_ref, s1_ref, s1b_ref, s2_ref, s2b_ref,
                  j1blk_ref, j1b_ref, j2half_ref, j2b_ref,
                  stem_ref, jb_ref):
    hs = _lrelu(
        jnp.dot(gs_ref[...], s1_ref[...], preferred_element_type=jnp.float32)
        + s1b_ref[...])
    stem_ref[...] = (
        jnp.dot(hs, s2_ref[...], preferred_element_type=jnp.float32)
        + s2b_ref[...])
    hj = _lrelu(
        jnp.dot(gj_ref[...], j1blk_ref[...], preferred_element_type=jnp.float32)
        + j1b_ref[...])
    jb_ref[...] = (
        jnp.dot(hj, j2half_ref[...], preferred_element_type=jnp.float32)
        + j2b_ref[...])


def _readouts(gs, gj, s1, s1b, s2, s2b, j1blk, j1b2, j2half, j2b):
    nstem = gs.shape[0]
    njb = gj.shape[0]
    nout = s2.shape[1]
    return pl.pallas_call(
        _readout_body,
        out_shape=(jax.ShapeDtypeStruct((nstem, nout), jnp.float32),
                   jax.ShapeDtypeStruct((njb, 1), jnp.float32)),
    )(gs, gj, s1, s1b.reshape(1, -1), s2, s2b.reshape(1, -1),
      j1blk, j1b2.reshape(1, -1), j2half, j2b.reshape(1, 1))


def _set2set_body(out_ref, batch_ref, gsum_ref, lwq_ref, lwr_ref, lb_ref,
                  so_ref, *, nb, dim):
    out = out_ref[...]
    gates = gsum_ref[...]
    i_ = jax.nn.sigmoid(gates[:, :dim])
    g_ = jnp.tanh(gates[:, 2 * dim:3 * dim])
    o_ = jax.nn.sigmoid(gates[:, 3 * dim:])
    cl = i_ * g_
    qvec = o_ * jnp.tanh(cl)
    e = jnp.dot(out, qvec.T, preferred_element_type=jnp.float32)
    bvec = batch_ref[...]
    iota = jax.lax.broadcasted_iota(jnp.int32, (1, nb), 1)
    oh_bool = bvec == iota
    oh = oh_bool.astype(jnp.float32)
    neg = jnp.float32(-1e30)
    masked = jnp.where(oh_bool, e, neg)
    emax = jnp.max(masked, axis=0, keepdims=True)
    emax = jnp.where(emax < -1e29, 0.0, emax)
    e_shift = e - jnp.dot(oh, emax.T, preferred_element_type=jnp.float32)
    a = jnp.exp(e_shift)
    asum = jnp.sum(oh * a, axis=0, keepdims=True)
    asum = jnp.clip(asum, 1e-12, None)
    a = a / jnp.dot(oh, asum.T, preferred_element_type=jnp.float32)
    ao = a * out
    rvec = jax.lax.dot_general(oh, ao, (((0,), (0,)), ((), ())),
                               preferred_element_type=jnp.float32)
    so_ref[...] = (
        jnp.dot(rvec, lwr_ref[...], preferred_element_type=jnp.float32)
        + jnp.dot(qvec, lwq_ref[...], preferred_element_type=jnp.float32)
        + lb_ref[...])


def _set2set(out, batch, gsum, lwq, lwr, lb, nb):
    n, dim = out.shape
    return pl.pallas_call(
        functools.partial(_set2set_body, nb=nb, dim=dim),
        out_shape=jax.ShapeDtypeStruct((nb, 2), jnp.float32),
    )(out, batch.reshape(n, 1), gsum.reshape(1, -1), lwq, lwr,
      lb.reshape(1, 2))


def kernel(x, edge_attr, params, edge_index, stem_atmidx, jbond_atmidx, batch):
    p = params
    n, _ = x.shape
    e = edge_attr.shape[0]
    dim = p['lin0_w'].shape[1]
    nb = 256

    src = edge_index[0]
    dst = edge_index[1]

    w2r = p['en2_w'].reshape(dim * dim, dim)
    rexp = jnp.repeat(jnp.eye(dim, dtype=jnp.float32), dim, axis=1)
    tmat = jnp.tile(jnp.eye(dim, dtype=jnp.float32), (1, dim))
    b2 = p['en2_b'].reshape(dim, dim)
    wih_t = p['gru_wih'].T
    whh_t = p['gru_whh'].T
    j1 = p['j1_w']
    j1blk = jnp.block([[j1, jnp.zeros_like(j1)], [jnp.zeros_like(j1), j1]])
    j1b2 = jnp.concatenate([p['j1_b'], p['j1_b']])
    j2half = 0.5 * jnp.concatenate([p['j2_w'], p['j2_w']], axis=0)
    lwq = p['lout_w'][:dim]
    lwr = p['lout_w'][dim:]
    gsum = p['lstm_bih'] + p['lstm_bhh']

    te = 2000 if e % 2000 == 0 else e

    out = _encode_nodes(x, p['lin0_w'], p['lin0_b'])
    h = out
    eh = _encode_edges(edge_attr, p['en1_w'], p['en1_b'], min(te * 10, e))

    cp = _sc_scatter_add(jnp.ones((e, dim), jnp.float32), dst, n, 1000)

    for _ in range(6):
        u = _sc_gather(out, src, 1000)
        msg = _messages(u, eh, rexp, tmat, w2r, b2, te)
        ap = _sc_scatter_add(msg, dst, n, 1000)
        out, h = _gru(out, h, ap, cp, p['conv_root'],
                      p['conv_b'].reshape(1, dim), wih_t, whh_t,
                      p['gru_bih'], p['gru_bhh'])

    gth = _sc_gather(
        out, jnp.concatenate([stem_atmidx, jbond_atmidx.reshape(-1)]), 128)
    gs = gth[:stem_atmidx.shape[0]]
    gj = gth[stem_atmidx.shape[0]:].reshape(-1, 2 * dim)

    stem_preds, jb = _readouts(gs, gj, p['s1_w'], p['s1_b'], p['s2_w'],
                               p['s2_b'], j1blk, j1b2, j2half, p['j2_b'])
    jbond_preds = jb.reshape(-1)

    scalar_outs = _set2set(out, batch, gsum, lwq, lwr, p['lout_b'], nb)
    return scalar_outs, stem_preds, jbond_preds
